# baseline (speedup 1.0000x reference)
import functools

import jax
import jax.numpy as jnp
from jax import lax
from jax.experimental import pallas as pl
from jax.experimental.pallas import tpu as pltpu
from jax.experimental.pallas import tpu_sc as plsc

_B = 16384
_H = 50
_V = 1000
_D = 128
_NB = _B * _H

_NC = 2
_NS = 16
_NW = _NC * _NS
_BPW = _NB // _NW
_IC = 256
_NCH = _BPW // _IC
_SG = 4
_NSF = _IC // (4 * _SG)

def _transform_body(e_ref, w_ref, b_ref, t_ref):
    prod = lax.dot_general(
        e_ref[...], w_ref[...], (((1,), (1,)), ((), ())),
        preferred_element_type=jnp.float32,
        precision=lax.Precision.HIGHEST)
    t_ref[...] = jnp.maximum(prod + b_ref[...], 0.0)


def _transform_table(emb_table, W, b):
    return pl.pallas_call(
        _transform_body,
        out_shape=jax.ShapeDtypeStruct((_V, _D), jnp.float32),
    )(emb_table, W, b.reshape(1, _D))


def _gather_body(table_hbm, idx_hbm, out_hbm, tbl, ib0, ib1,
                 st0, st1, st2, st3, isem0, isem1, osem):
    wid = lax.axis_index("s") * _NC + lax.axis_index("c")
    rbase = wid * _BPW
    stages = (st0, st1, st2, st3)
    cols = [lax.iota(jnp.int32, 16) + 16 * j for j in range(_D // 16)]

    pltpu.sync_copy(table_hbm, tbl)

    def out_sl(row0, b):
        return out_hbm.at[pl.ds(rbase + row0 + b * _SG, _SG), :]

    def superflush(ib, row0, loc0):
        ids = ib[pl.ds(loc0, 16)]
        for b in range(4):
            st = stages[b]
            pltpu.make_async_copy(st, out_sl(row0, b), osem).wait()
            for r in range(_SG):
                fbase = jnp.broadcast_to(ids[b * _SG + r] * _D, (16,))
                for j in range(_D // 16):
                    st[r, pl.ds(j * 16, 16)] = plsc.load_gather(
                        tbl, [fbase + cols[j]])
            pltpu.async_copy(st, out_sl(row0, b), osem)

    def do_chunk(row0, ib, isem, pf_row0, pf_ib, pf_isem):
        pltpu.make_async_copy(
            idx_hbm.at[pl.ds(rbase + row0, _IC)], ib, isem).wait()
        pf_off = jnp.minimum(rbase + pf_row0, _NB - _IC)
        pltpu.async_copy(idx_hbm.at[pl.ds(pf_off, _IC)], pf_ib, pf_isem)

        def sf_body(sf, carry):
            loc0 = sf * (4 * _SG)
            superflush(ib, row0 + loc0, loc0)
            return carry

        lax.fori_loop(0, _NSF, sf_body, 0)

    pltpu.async_copy(idx_hbm.at[pl.ds(rbase, _IC)], ib0, isem0)
    for b in range(4):
        pltpu.async_copy(stages[b], out_sl(0, b), osem)

    def pair_body(j, carry):
        row_a = (2 * j) * _IC
        do_chunk(row_a, ib0, isem0, row_a + _IC, ib1, isem1)
        do_chunk(row_a + _IC, ib1, isem1, row_a + 2 * _IC, ib0, isem0)
        return carry

    lax.fori_loop(0, _NCH // 2, pair_body, 0)

    last = _BPW - 4 * _SG
    for b in range(4):
        pltpu.make_async_copy(stages[b], out_sl(last, b), osem).wait()
    pltpu.make_async_copy(
        idx_hbm.at[pl.ds(_NB - _IC, _IC)], ib0, isem0).wait()


def _gather(table, idx):
    mesh = plsc.VectorSubcoreMesh(core_axis_name="c", subcore_axis_name="s")
    run = functools.partial(
        pl.kernel,
        mesh=mesh,
        compiler_params=pltpu.CompilerParams(needs_layout_passes=False),
        out_type=jax.ShapeDtypeStruct((_NB, _D), jnp.float32),
        scratch_types=[
            pltpu.VMEM((_V * _D,), jnp.float32),
            pltpu.VMEM((_IC,), jnp.int32),
            pltpu.VMEM((_IC,), jnp.int32),
            pltpu.VMEM((_SG, _D), jnp.float32),
            pltpu.VMEM((_SG, _D), jnp.float32),
            pltpu.VMEM((_SG, _D), jnp.float32),
            pltpu.VMEM((_SG, _D), jnp.float32),
            pltpu.SemaphoreType.DMA,
            pltpu.SemaphoreType.DMA,
            pltpu.SemaphoreType.DMA,
        ],
    )(_gather_body)
    return run(table, idx)


def kernel(industry_ids, emb_table, W, b):
    table = _transform_table(emb_table, W, b)
    idx = industry_ids.reshape(_NB).astype(jnp.int32)
    out = _gather(table.reshape(_V * _D), idx)
    return out.reshape(_B, _H, _D)


# speedup vs baseline: 2.0621x; 2.0621x over previous
import functools

import jax
import jax.numpy as jnp
from jax import lax
from jax.experimental import pallas as pl
from jax.experimental.pallas import tpu as pltpu
from jax.experimental.pallas import tpu_sc as plsc

_B = 16384
_H = 50
_V = 1000
_D = 128
_NB = _B * _H

_NC = 2
_NS = 16
_NW = _NC * _NS
_BPT = _B // _NW
_IPT = _BPT * _H
_CB = 4
_CR = _CB * _H
_NCH = _BPT // _CB


def _transform_body(e_ref, w_ref, b_ref, t_ref):
    prod = lax.dot_general(
        e_ref[...], w_ref[...], (((1,), (1,)), ((), ())),
        preferred_element_type=jnp.float32,
        precision=lax.Precision.HIGHEST)
    t_ref[...] = jnp.maximum(prod + b_ref[...], 0.0)


def _transform_table(emb_table, W, b):
    return pl.pallas_call(
        _transform_body,
        out_shape=jax.ShapeDtypeStruct((_V, _D), jnp.float32),
    )(emb_table, W, b.reshape(1, _D))


def _gather_body(table_hbm, idx_hbm, out_hbm, idx_v, stga, stgb,
                 ga, gb, oa, ob):
    wid = lax.axis_index("s") * _NC + lax.axis_index("c")
    batch0 = wid * _BPT
    pltpu.sync_copy(idx_hbm.at[pl.ds(wid * _IPT, _IPT)], idx_v)

    def gather(c, stg, sem):
        off = jnp.minimum(c * _CR, _IPT - _CR)
        return pltpu.make_async_copy(
            table_hbm.at[idx_v.at[pl.ds(off, _CR)]], stg, sem)

    def fire_outs(c, stg, sem):
        for k in range(_CB):
            pltpu.async_copy(
                stg.at[pl.ds(k * _H, _H)],
                out_hbm.at[batch0 + c * _CB + k], sem)

    def drain_outs(c, stg, sem):
        for k in range(_CB):
            pltpu.make_async_copy(
                stg.at[pl.ds(k * _H, _H)],
                out_hbm.at[batch0 + c * _CB + k], sem).wait()

    gather(0, stga, ga).start()

    def body(j, carry):
        a = 2 * j
        gather(a, stga, ga).wait()
        gather(a + 1, stgb, gb).start()
        fire_outs(a, stga, oa)
        gather(a + 1, stgb, gb).wait()
        drain_outs(a, stga, oa)
        gather(a + 2, stga, ga).start()
        fire_outs(a + 1, stgb, ob)
        drain_outs(a + 1, stgb, ob)
        return carry

    lax.fori_loop(0, _NCH // 2, body, 0)
    gather(_NCH, stga, ga).wait()


def _gather(table, idx):
    mesh = plsc.VectorSubcoreMesh(core_axis_name="c", subcore_axis_name="s")
    run = functools.partial(
        pl.kernel,
        mesh=mesh,
        compiler_params=pltpu.CompilerParams(needs_layout_passes=False),
        out_type=jax.ShapeDtypeStruct((_B, _H, _D), jnp.float32),
        scratch_types=[
            pltpu.VMEM((_IPT,), jnp.int32),
            pltpu.VMEM((_CR, _D), jnp.float32),
            pltpu.VMEM((_CR, _D), jnp.float32),
            pltpu.SemaphoreType.DMA,
            pltpu.SemaphoreType.DMA,
            pltpu.SemaphoreType.DMA,
            pltpu.SemaphoreType.DMA,
        ],
    )(_gather_body)
    return run(table, idx)


def kernel(industry_ids, emb_table, W, b):
    table = _transform_table(emb_table, W, b)
    idx = industry_ids.reshape(_NB).astype(jnp.int32)
    return _gather(table, idx)


# speedup vs baseline: 3.2557x; 1.5788x over previous
import functools

import jax
import jax.numpy as jnp
from jax import lax
from jax.experimental import pallas as pl
from jax.experimental.pallas import tpu as pltpu
from jax.experimental.pallas import tpu_sc as plsc

_B = 16384
_H = 50
_V = 1000
_D = 128
_NB = _B * _H

_NC = 2
_NS = 16
_NW = _NC * _NS
_BPT = _B // _NW
_IPT = _BPT * _H
_CB = 4
_CR = _CB * _H
_NCH = _BPT // _CB


def _transform_body(e_ref, w_ref, b_ref, t_ref):
    prod = lax.dot_general(
        e_ref[...], w_ref[...], (((1,), (1,)), ((), ())),
        preferred_element_type=jnp.float32,
        precision=lax.Precision.HIGHEST)
    t_ref[...] = jnp.maximum(prod + b_ref[...], 0.0)


def _transform_table(emb_table, W, b):
    return pl.pallas_call(
        _transform_body,
        out_shape=jax.ShapeDtypeStruct((_V, _D), jnp.float32),
    )(emb_table, W, b.reshape(1, _D))


def _gather_body(table_hbm, idx_hbm, out_hbm, tbl_sh, idx_v, stga, stgb,
                 ga, gb, oa, ob):
    wid = lax.axis_index("s") * _NC + lax.axis_index("c")
    batch0 = wid * _BPT
    @pl.when(lax.axis_index("s") == 0)
    def _():
        pltpu.sync_copy(table_hbm, tbl_sh)
    pltpu.sync_copy(idx_hbm.at[pl.ds(wid * _IPT, _IPT)], idx_v)
    plsc.subcore_barrier()

    def gather(c, stg, sem):
        off = jnp.minimum(c * _CR, _IPT - _CR)
        return pltpu.make_async_copy(
            tbl_sh.at[idx_v.at[pl.ds(off, _CR)]], stg, sem)

    def fire_outs(c, stg, sem):
        for k in range(_CB):
            pltpu.async_copy(
                stg.at[pl.ds(k * _H, _H)],
                out_hbm.at[batch0 + c * _CB + k], sem)

    def drain_outs(c, stg, sem):
        for k in range(_CB):
            pltpu.make_async_copy(
                stg.at[pl.ds(k * _H, _H)],
                out_hbm.at[batch0 + c * _CB + k], sem).wait()

    gather(0, stga, ga).start()

    def body(j, carry):
        a = 2 * j
        gather(a, stga, ga).wait()
        gather(a + 1, stgb, gb).start()
        fire_outs(a, stga, oa)
        gather(a + 1, stgb, gb).wait()
        drain_outs(a, stga, oa)
        gather(a + 2, stga, ga).start()
        fire_outs(a + 1, stgb, ob)
        drain_outs(a + 1, stgb, ob)
        return carry

    lax.fori_loop(0, _NCH // 2, body, 0)
    gather(_NCH, stga, ga).wait()


def _gather(table, idx):
    mesh = plsc.VectorSubcoreMesh(core_axis_name="c", subcore_axis_name="s")
    run = functools.partial(
        pl.kernel,
        mesh=mesh,
        compiler_params=pltpu.CompilerParams(needs_layout_passes=False),
        out_type=jax.ShapeDtypeStruct((_B, _H, _D), jnp.float32),
        scratch_types=[
            pltpu.VMEM_SHARED((_V, _D), jnp.float32),
            pltpu.VMEM((_IPT,), jnp.int32),
            pltpu.VMEM((_CR, _D), jnp.float32),
            pltpu.VMEM((_CR, _D), jnp.float32),
            pltpu.SemaphoreType.DMA,
            pltpu.SemaphoreType.DMA,
            pltpu.SemaphoreType.DMA,
            pltpu.SemaphoreType.DMA,
        ],
    )(_gather_body)
    return run(table, idx)


def kernel(industry_ids, emb_table, W, b):
    table = _transform_table(emb_table, W, b)
    idx = industry_ids.reshape(_NB).astype(jnp.int32)
    return _gather(table, idx)


# speedup vs baseline: 8.4408x; 2.5926x over previous
import functools

import jax
import jax.numpy as jnp
from jax import lax
from jax.experimental import pallas as pl
from jax.experimental.pallas import tpu as pltpu
from jax.experimental.pallas import tpu_sc as plsc

_B = 16384
_H = 50
_V = 1000
_D = 128

_NC = 2
_NS = 16
_NW = _NC * _NS
_BPT = _B // _NW
_CR = 256
_NCH = _H * 2


def _transform_body(e_ref, w_ref, b_ref, t_ref):
    prod = lax.dot_general(
        e_ref[...], w_ref[...], (((1,), (1,)), ((), ())),
        preferred_element_type=jnp.float32,
        precision=lax.Precision.HIGHEST)
    t_ref[...] = jnp.maximum(prod + b_ref[...], 0.0)


def _transform_table(emb_table, W, b):
    return pl.pallas_call(
        _transform_body,
        out_shape=jax.ShapeDtypeStruct((_V, _D), jnp.float32),
    )(emb_table, W, b.reshape(1, _D))


def _gather_body(table_hbm, idx_hbm, out_hbm, tbl_sh, idx_v, stga, stgb,
                 isem, ga, gb, oa, ob):
    wid = lax.axis_index("s") * _NC + lax.axis_index("c")
    b0 = wid * _BPT
    @pl.when(lax.axis_index("s") == 0)
    def _():
        pltpu.sync_copy(table_hbm, tbl_sh)

    def idx_dma(h):
        return pltpu.make_async_copy(
            idx_hbm.at[h, pl.ds(b0, _BPT)],
            idx_v.at[pl.ds(h * _BPT, _BPT)], isem)

    def fire_idx(h, carry):
        idx_dma(h).start()
        return carry

    def drain_idx(h, carry):
        idx_dma(h).wait()
        return carry

    lax.fori_loop(0, _H, fire_idx, 0)
    lax.fori_loop(0, _H, drain_idx, 0)
    plsc.subcore_barrier()

    def gather(h, half, stg, sem):
        h = jnp.minimum(h, _H - 1)
        return pltpu.make_async_copy(
            tbl_sh.at[idx_v.at[pl.ds(h * (2 * _CR) + half * _CR, _CR)]],
            stg, sem)

    def out_copy(h, half, stg, sem):
        return pltpu.make_async_copy(
            stg, out_hbm.at[h, pl.ds(b0 + half * _CR, _CR), :], sem)

    gather(0, 0, stga, ga).start()

    def body(h, carry):
        gather(h, 0, stga, ga).wait()
        gather(h, 1, stgb, gb).start()
        out_copy(h, 0, stga, oa).start()
        gather(h, 1, stgb, gb).wait()
        out_copy(h, 0, stga, oa).wait()
        gather(h + 1, 0, stga, ga).start()
        out_copy(h, 1, stgb, ob).start()
        out_copy(h, 1, stgb, ob).wait()
        return carry

    lax.fori_loop(0, _H, body, 0)
    gather(_H, 0, stga, ga).wait()


def _gather(table, idx_t):
    mesh = plsc.VectorSubcoreMesh(core_axis_name="c", subcore_axis_name="s")
    run = functools.partial(
        pl.kernel,
        mesh=mesh,
        compiler_params=pltpu.CompilerParams(needs_layout_passes=False),
        out_type=jax.ShapeDtypeStruct((_H, _B, _D), jnp.float32),
        scratch_types=[
            pltpu.VMEM_SHARED((_V, _D), jnp.float32),
            pltpu.VMEM((_H * _BPT,), jnp.int32),
            pltpu.VMEM((_CR, _D), jnp.float32),
            pltpu.VMEM((_CR, _D), jnp.float32),
            pltpu.SemaphoreType.DMA,
            pltpu.SemaphoreType.DMA,
            pltpu.SemaphoreType.DMA,
            pltpu.SemaphoreType.DMA,
            pltpu.SemaphoreType.DMA,
        ],
    )(_gather_body)
    return run(table, idx_t)


def kernel(industry_ids, emb_table, W, b):
    table = _transform_table(emb_table, W, b)
    idx_t = industry_ids.astype(jnp.int32).T
    out_t = _gather(table, idx_t)
    return jnp.transpose(out_t, (1, 0, 2))
